# baseline (speedup 1.0000x reference)
import functools

import jax
import jax.numpy as jnp
from jax import lax
from jax.experimental import pallas as pl
from jax.experimental.pallas import tpu as pltpu
from jax.experimental.pallas import tpu_sc as plsc

_D = 64
_NW = 32
_CHUNK = 128
_NB = 4
_SCALE = 8.0


def _emb_body(idx_hbm, table_hbm, out_hbm, idx_v, bufg, bufo, *sems):
    n_chunks = idx_v.shape[0]
    n_groups = n_chunks // _NB
    sem_g, sem_o = sems[:_NB], sems[_NB:]
    wid = lax.axis_index("s") * 2 + lax.axis_index("c")
    pltpu.sync_copy(idx_hbm.at[wid], idx_v)
    lane = lax.iota(jnp.int32, 16)

    def gather_start(j, b):
        pltpu.async_copy(table_hbm.at[idx_v.at[j]], bufg.at[b], sem_g[b])

    def gather_wait(j, b):
        pltpu.make_async_copy(table_hbm.at[idx_v.at[j]], bufg.at[b],
                              sem_g[b]).wait()

    def out_start(j, b):
        for a in range(8):
            pltpu.async_copy(bufo.at[b].at[a],
                             out_hbm.at[j * 256 + a * 32 + wid], sem_o[b])

    def out_wait(j, b):
        for a in range(8):
            pltpu.make_async_copy(bufo.at[b].at[a],
                                  out_hbm.at[j * 256 + a * 32 + wid],
                                  sem_o[b]).wait()

    def transpose_chunk(b):
        src, dst = bufg.at[b], bufo.at[b]

        def feat(q, c):
            a, r = q // 8, q % 8
            col = jnp.full((16,), 8, jnp.int32) * a + r
            for cg in range(8):
                vals = plsc.load_gather(src, [cg * 16 + lane, col])
                dst[a, pl.ds(r * 128 + cg * 16, 16)] = vals * _SCALE
            return c

        lax.fori_loop(0, 64, feat, 0, unroll=2)

    def group(g, first, fire):
        for b in range(_NB):
            j = g * _NB + b
            gather_wait(j, b)
            if not first:
                out_wait(j, b)
            transpose_chunk(b)
            if fire:
                gather_start(j + _NB, b)
            out_start(j, b)

    for b in range(_NB):
        gather_start(b, b)
    group(0, first=True, fire=True)
    lax.fori_loop(1, n_groups - 1,
                  lambda g, c: (group(g, first=False, fire=True), c)[1], 0)
    group(n_groups - 1, first=False, fire=False)
    for b in range(_NB):
        out_wait((n_groups - 1) * _NB + b, b)


def kernel(x, emb_weight):
    b0, b1 = x.shape
    total = b0 * b1
    n_chunks = b1
    assert b0 == _NW * _CHUNK and total == _NW * _CHUNK * n_chunks
    xi = x.reshape(_NW, _CHUNK, n_chunks).transpose(0, 2, 1).astype(jnp.int32)

    mesh = plsc.VectorSubcoreMesh(core_axis_name="c", subcore_axis_name="s")
    run = functools.partial(
        pl.kernel,
        out_type=jax.ShapeDtypeStruct((n_chunks * 8 * _NW, 1024), jnp.float32),
        mesh=mesh,
        scratch_types=[
            pltpu.VMEM((n_chunks, _CHUNK), jnp.int32),
            pltpu.VMEM((_NB, _CHUNK, _D), jnp.float32),
            pltpu.VMEM((_NB, 8, 1024), jnp.float32),
        ] + [pltpu.SemaphoreType.DMA] * (2 * _NB),
        compiler_params=pltpu.CompilerParams(use_tc_tiling_on_sc=False,
                                             needs_layout_passes=False),
    )(_emb_body)
    out2 = run(xi, emb_weight)
    out5 = out2.reshape(n_chunks, 8, _NW, 8, _CHUNK)
    return out5.transpose(2, 4, 0, 1, 3).reshape(b0, b1, _D)
